# baseline (speedup 1.0000x reference)
import functools

import jax
import jax.numpy as jnp
from jax import lax
from jax.experimental import pallas as pl
from jax.experimental.pallas import tpu as pltpu
from jax.experimental.pallas import tpu_sc as plsc

_K = 4
_N = 10000
_E = 320000
_D = 128
_DS = _D // _K

_NC = 2
_NS = 16
_NW = _NC * _NS
_CH = 128
_C = 80
_IB = 4
_RB = 2
_EPAD = _NW * _C * _CH
_NACC = 10112
_RPT = _NACC // _NS


def _sc_scatter_add(x, idxp, zeros_tile):
    mesh = plsc.VectorSubcoreMesh(core_axis_name="c", subcore_axis_name="s")

    @functools.partial(
        pl.kernel,
        out_type=jax.ShapeDtypeStruct((_NC * _NACC, _D), jnp.float32),
        mesh=mesh,
        scratch_types=(
            [pltpu.VMEM((2, _CH), jnp.int32) for _ in range(_IB)]
            + [pltpu.VMEM((_CH, _D), jnp.float32) for _ in range(_RB)]
            + [pltpu.VMEM_SHARED((_NACC, _D), jnp.float32)]
            + [pltpu.SemaphoreType.DMA for _ in range(_IB + _RB)]),
    )
    def k(x_hbm, idx_hbm, z_hbm, out_hbm, *rest):
        idxb = rest[:_IB]
        rows = rest[_IB:_IB + _RB]
        acc = rest[_IB + _RB]
        isem = rest[_IB + _RB + 1:_IB + _RB + 1 + _IB]
        gsem = rest[_IB + _RB + 1 + _IB:]
        c = lax.axis_index("c")
        s = lax.axis_index("s")
        wid = c * _NS + s
        base = wid * (_C + 2)
        pltpu.sync_copy(z_hbm, acc.at[pl.ds(s * _RPT, _RPT)])
        plsc.subcore_barrier()

        def fetch(chunk, slot):
            pltpu.async_copy(idx_hbm.at[base + chunk], idxb[slot],
                             isem[slot])

        def fetch_wait(chunk, slot):
            pltpu.make_async_copy(idx_hbm.at[base + chunk], idxb[slot],
                                  isem[slot]).wait()

        def gather(slot, rslot):
            pltpu.async_copy(x_hbm.at[idxb[slot].at[0]], rows[rslot],
                             gsem[rslot])

        def gather_wait(slot, rslot):
            pltpu.make_async_copy(x_hbm.at[idxb[slot].at[0]], rows[rslot],
                                  gsem[rslot]).wait()

        fetch(0, 0)
        fetch(1, 1)
        fetch_wait(0, 0)
        gather(0, 0)

        def step(t, carry):
            j = t * _IB
            for b in range(_IB):
                tt = j + b
                fetch(tt + 2, (b + 2) % _IB)
                fetch_wait(tt + 1, (b + 1) % _IB)
                gather((b + 1) % _IB, (b + 1) % _RB)
                gather_wait(b, b % _RB)
                pltpu.sync_copy(rows[b % _RB], acc.at[idxb[b].at[1]],
                                add=True)
            return carry

        lax.fori_loop(0, _C // _IB, step, 0)
        gather_wait(0, 0)
        fetch_wait(_C + 1, 1)
        plsc.subcore_barrier()
        pltpu.sync_copy(acc.at[pl.ds(s * _RPT, _RPT)],
                        out_hbm.at[pl.ds(c * _NACC + s * _RPT, _RPT)])

    return k(x, idxp, zeros_tile)


def _mlp_body(scale_ref, x_ref, a_ref, b_ref, w1_ref, c1_ref, w2_ref,
              c2_ref, wl_ref, cl_ref, o_ref):
    h = x_ref[...] + a_ref[...] + b_ref[...]
    r = jnp.maximum(
        jnp.dot(h, w1_ref[...], preferred_element_type=jnp.float32)
        + c1_ref[...], 0.0)
    z = (jnp.dot(r, w2_ref[...], preferred_element_type=jnp.float32)
         + c2_ref[...])
    lin = (jnp.dot(x_ref[...], wl_ref[...],
                   preferred_element_type=jnp.float32) + cl_ref[...])
    o_ref[...] = z + scale_ref[0, 0] * lin


def _block_diag(w):
    k, a, b = w.shape
    out = jnp.zeros((k * a, k * b), w.dtype)
    for i in range(k):
        out = out.at[i * a:(i + 1) * a, i * b:(i + 1) * b].set(w[i])
    return out


def kernel(x, edge_index, edge_weights, edge_attr, with_linear, W1, b1, W2,
           b2, Wl, bl):
    src = edge_index[0]
    dst = edge_index[1]
    pad = _EPAD - _E
    srcp = jnp.concatenate([src, jnp.zeros((pad,), jnp.int32)])
    dstp = jnp.concatenate([dst, jnp.full((pad,), _N, jnp.int32)])
    srcp = srcp.reshape(_NW, _C, _CH)
    dstp = dstp.reshape(_NW, _C, _CH)
    idxp = jnp.stack([srcp, dstp], axis=2)
    idxp = jnp.concatenate(
        [idxp, jnp.zeros((_NW, 2, 2, _CH), jnp.int32)], axis=1)
    idxp = idxp.reshape(_NW * (_C + 2), 2, _CH)
    zeros_tile = jnp.zeros((_RPT, _D), jnp.float32)

    parts = _sc_scatter_add(x, idxp, zeros_tile)
    agg0 = parts[:_N]
    agg1 = parts[_NACC:_NACC + _N]

    bd1 = _block_diag(W1)
    bd2 = _block_diag(W2)
    bdl = _block_diag(Wl[:, :_DS, :_DS])
    c1 = b1.reshape(1, _D)
    c2 = b2.reshape(1, _D)
    cl = bl[:, :_DS].reshape(1, _D)
    scale = jnp.where(with_linear != 0, 1.0, 0.0).astype(
        jnp.float32).reshape(1, 1)

    blk = 1000
    grid = _N // blk
    full = pl.BlockSpec((_D, _D), lambda i: (0, 0))
    bias = pl.BlockSpec((1, _D), lambda i: (0, 0))
    rows = pl.BlockSpec((blk, _D), lambda i: (i, 0))
    out = pl.pallas_call(
        _mlp_body,
        grid=(grid,),
        in_specs=[pl.BlockSpec((1, 1), lambda i: (0, 0)),
                  rows, rows, rows, full, bias, full, bias, full, bias],
        out_specs=rows,
        out_shape=jax.ShapeDtypeStruct((_N, _D), jnp.float32),
    )(scale, x, agg0, agg1, bd1, c1, bd2, c2, bdl, cl)
    return out
